# Initial kernel scaffold; baseline (speedup 1.0000x reference)
import jax
import jax.numpy as jnp
from jax.experimental import pallas as pl


def kernel(x, degrees, degree_table):
    raise NotImplementedError("write your pallas kernel here")


# speedup vs baseline: 1.3495x; 1.3495x over previous
import functools

import jax
import jax.numpy as jnp
from jax import lax
from jax.experimental import pallas as pl
from jax.experimental.pallas import tpu as pltpu
from jax.experimental.pallas import tpu_sc as plsc

N = 100000
D = 128
NUM_DEGREE = 512

NC = 2
NS = 16
NW = NC * NS

CH = 80
NCHUNK = N // CH
ITERS = (NCHUNK + NW - 1) // NW


def _sc_body(x_hbm, deg_hbm, tab_hbm, out_hbm, deg_v, xbuf, gbuf, sem):
    wid = lax.axis_index("s") * NC + lax.axis_index("c")

    def chunk_body(t, carry):
        cid = t * NW + wid

        @pl.when(cid < NCHUNK)
        def _():
            base = cid * CH
            pltpu.sync_copy(deg_hbm.at[pl.ds(base, CH)], deg_v)
            pltpu.async_copy(tab_hbm.at[deg_v], gbuf, sem).wait()
            pltpu.sync_copy(x_hbm.at[pl.ds(base, CH), :], xbuf)

            def row_body(r, c2):
                for dcol in range(D // 16):
                    sl = pl.ds(dcol * 16, 16)
                    gbuf[r, sl] = gbuf[r, sl] + xbuf[r, sl]
                return c2

            lax.fori_loop(0, CH, row_body, 0)
            pltpu.sync_copy(gbuf, out_hbm.at[pl.ds(base, CH), :])

        return carry

    lax.fori_loop(0, ITERS, chunk_body, 0)


@jax.jit
def _run(x, degrees_i32, degree_table):
    kern = pl.kernel(
        _sc_body,
        out_type=jax.ShapeDtypeStruct((N, D), jnp.float32),
        mesh=plsc.VectorSubcoreMesh(core_axis_name="c", subcore_axis_name="s"),
        scratch_types=[
            pltpu.VMEM((CH,), jnp.int32),
            pltpu.VMEM((CH, D), jnp.float32),
            pltpu.VMEM((CH, D), jnp.float32),
            pltpu.SemaphoreType.DMA,
        ],
    )
    return kern(x, degrees_i32, degree_table)


def kernel(x, degrees, degree_table):
    return _run(x, degrees.astype(jnp.int32), degree_table)


# speedup vs baseline: 2.3634x; 1.7513x over previous
import jax
import jax.numpy as jnp
from jax import lax
from jax.experimental import pallas as pl
from jax.experimental.pallas import tpu as pltpu
from jax.experimental.pallas import tpu_sc as plsc

N = 100000
D = 128
NUM_DEGREE = 512

NC = 2
NS = 16
NW = NC * NS

CH = 80
NCHUNK = N // CH
ITERS = (NCHUNK + NW - 1) // NW
OUTER = ITERS // 2


def _sc_body(x_hbm, deg_hbm, tab_hbm, out_hbm,
             deg0, deg1, xb0, xb1, gb0, gb1,
             gs0, gs1, xs0, xs1, os0, os1):
    wid = lax.axis_index("s") * NC + lax.axis_index("c")
    degs = (deg0, deg1)
    xbs = (xb0, xb1)
    gbs = (gb0, gb1)
    gss = (gs0, gs1)
    xss = (xs0, xs1)
    oss = (os0, os1)

    def issue(cid, b, owait):
        @pl.when(cid < NCHUNK)
        def _():
            base = cid * CH

            @pl.when(owait)
            def _():
                pltpu.make_async_copy(
                    gbs[b], out_hbm.at[pl.ds(0, CH), :], oss[b]).wait()

            pltpu.sync_copy(deg_hbm.at[pl.ds(base, CH)], degs[b])
            pltpu.async_copy(tab_hbm.at[degs[b]], gbs[b], gss[b])
            pltpu.async_copy(x_hbm.at[pl.ds(base, CH), :], xbs[b], xss[b])

    def work(cid, b):
        @pl.when(cid < NCHUNK)
        def _():
            base = cid * CH
            pltpu.make_async_copy(tab_hbm.at[degs[b]], gbs[b], gss[b]).wait()
            pltpu.make_async_copy(
                x_hbm.at[pl.ds(base, CH), :], xbs[b], xss[b]).wait()

            def row_body(r, c2):
                for dcol in range(D // 16):
                    sl = pl.ds(dcol * 16, 16)
                    gbs[b][r, sl] = gbs[b][r, sl] + xbs[b][r, sl]
                return c2

            lax.fori_loop(0, CH, row_body, 0)
            pltpu.async_copy(gbs[b], out_hbm.at[pl.ds(base, CH), :], oss[b])

    issue(wid, 0, False)

    def outer_body(i, carry):
        t0 = 2 * i * NW + wid
        t1 = t0 + NW
        t2 = t1 + NW
        issue(t1, 1, i >= 1)
        work(t0, 0)
        issue(t2, 0, True)
        work(t1, 1)
        return carry

    lax.fori_loop(0, OUTER, outer_body, 0)

    pltpu.make_async_copy(gbs[0], out_hbm.at[pl.ds(0, CH), :], oss[0]).wait()
    pltpu.make_async_copy(gbs[1], out_hbm.at[pl.ds(0, CH), :], oss[1]).wait()


@jax.jit
def _run(x, degrees_i32, degree_table):
    kern = pl.kernel(
        _sc_body,
        out_type=jax.ShapeDtypeStruct((N, D), jnp.float32),
        mesh=plsc.VectorSubcoreMesh(core_axis_name="c", subcore_axis_name="s"),
        scratch_types=[
            pltpu.VMEM((CH,), jnp.int32),
            pltpu.VMEM((CH,), jnp.int32),
            pltpu.VMEM((CH, D), jnp.float32),
            pltpu.VMEM((CH, D), jnp.float32),
            pltpu.VMEM((CH, D), jnp.float32),
            pltpu.VMEM((CH, D), jnp.float32),
            pltpu.SemaphoreType.DMA,
            pltpu.SemaphoreType.DMA,
            pltpu.SemaphoreType.DMA,
            pltpu.SemaphoreType.DMA,
            pltpu.SemaphoreType.DMA,
            pltpu.SemaphoreType.DMA,
        ],
    )
    return kern(x, degrees_i32, degree_table)


def kernel(x, degrees, degree_table):
    return _run(x, degrees.astype(jnp.int32), degree_table)


# speedup vs baseline: 2.6693x; 1.1294x over previous
import jax
import jax.numpy as jnp
from jax import lax
from jax.experimental import pallas as pl
from jax.experimental.pallas import tpu as pltpu
from jax.experimental.pallas import tpu_sc as plsc

N = 100000
D = 128
NUM_DEGREE = 512

NC = 2
NS = 16
NW = NC * NS

CH = 80
NCHUNK = N // CH
ITERS = (NCHUNK + NW - 1) // NW
OUTER = ITERS // 2


def _sc_body(x_hbm, deg_hbm, tab_hbm, out_hbm,
             tab_sh, deg0, deg1, xb0, xb1, gb0, gb1,
             gs0, gs1, xs0, xs1, os0, os1):
    wid = lax.axis_index("s") * NC + lax.axis_index("c")
    sid = lax.axis_index("s")

    @pl.when(sid == 0)
    def _():
        pltpu.sync_copy(tab_hbm, tab_sh)

    plsc.subcore_barrier()
    degs = (deg0, deg1)
    xbs = (xb0, xb1)
    gbs = (gb0, gb1)
    gss = (gs0, gs1)
    xss = (xs0, xs1)
    oss = (os0, os1)

    def issue(cid, b, owait):
        @pl.when(cid < NCHUNK)
        def _():
            base = cid * CH

            @pl.when(owait)
            def _():
                pltpu.make_async_copy(
                    gbs[b], out_hbm.at[pl.ds(0, CH), :], oss[b]).wait()

            pltpu.sync_copy(deg_hbm.at[pl.ds(base, CH)], degs[b])
            pltpu.async_copy(tab_sh.at[degs[b]], gbs[b], gss[b])
            pltpu.async_copy(x_hbm.at[pl.ds(base, CH), :], xbs[b], xss[b])

    def work(cid, b):
        @pl.when(cid < NCHUNK)
        def _():
            base = cid * CH
            pltpu.make_async_copy(tab_sh.at[degs[b]], gbs[b], gss[b]).wait()
            pltpu.make_async_copy(
                x_hbm.at[pl.ds(base, CH), :], xbs[b], xss[b]).wait()

            def row_body(r, c2):
                for dcol in range(D // 16):
                    sl = pl.ds(dcol * 16, 16)
                    gbs[b][r, sl] = gbs[b][r, sl] + xbs[b][r, sl]
                return c2

            lax.fori_loop(0, CH, row_body, 0)
            pltpu.async_copy(gbs[b], out_hbm.at[pl.ds(base, CH), :], oss[b])

    issue(wid, 0, False)

    def outer_body(i, carry):
        t0 = 2 * i * NW + wid
        t1 = t0 + NW
        t2 = t1 + NW
        issue(t1, 1, i >= 1)
        work(t0, 0)
        issue(t2, 0, True)
        work(t1, 1)
        return carry

    lax.fori_loop(0, OUTER, outer_body, 0)

    pltpu.make_async_copy(gbs[0], out_hbm.at[pl.ds(0, CH), :], oss[0]).wait()
    pltpu.make_async_copy(gbs[1], out_hbm.at[pl.ds(0, CH), :], oss[1]).wait()


@jax.jit
def _run(x, degrees_i32, degree_table):
    kern = pl.kernel(
        _sc_body,
        out_type=jax.ShapeDtypeStruct((N, D), jnp.float32),
        mesh=plsc.VectorSubcoreMesh(core_axis_name="c", subcore_axis_name="s"),
        scratch_types=[
            pltpu.VMEM_SHARED((NUM_DEGREE, D), jnp.float32),
            pltpu.VMEM((CH,), jnp.int32),
            pltpu.VMEM((CH,), jnp.int32),
            pltpu.VMEM((CH, D), jnp.float32),
            pltpu.VMEM((CH, D), jnp.float32),
            pltpu.VMEM((CH, D), jnp.float32),
            pltpu.VMEM((CH, D), jnp.float32),
            pltpu.SemaphoreType.DMA,
            pltpu.SemaphoreType.DMA,
            pltpu.SemaphoreType.DMA,
            pltpu.SemaphoreType.DMA,
            pltpu.SemaphoreType.DMA,
            pltpu.SemaphoreType.DMA,
        ],
    )
    return kern(x, degrees_i32, degree_table)


def kernel(x, degrees, degree_table):
    return _run(x, degrees.astype(jnp.int32), degree_table)


# speedup vs baseline: 3.4316x; 1.2856x over previous
import jax
import jax.numpy as jnp
from jax import lax
from jax.experimental import pallas as pl
from jax.experimental.pallas import tpu as pltpu
from jax.experimental.pallas import tpu_sc as plsc

N = 100000
D = 128
NUM_DEGREE = 512

NC = 2
NS = 16
NW = NC * NS

G = 80
CH = 2 * G
NCHUNK = N // CH
ITERS = 20
BASE_CNT = NCHUNK // NW
EXTRA = NCHUNK % NW


def _sc_body(x_hbm, deg_hbm, tab_hbm, out_hbm,
             tab_sh, dega, xb0, xb1, gb0, gb1,
             gs0, gs1, xs0, xs1, os0, os1):
    wid = lax.axis_index("s") * NC + lax.axis_index("c")
    sid = lax.axis_index("s")
    xbs = (xb0, xb1)
    gbs = (gb0, gb1)
    gss = (gs0, gs1)
    xss = (xs0, xs1)
    oss = (os0, os1)

    @pl.when(sid == 0)
    def _():
        pltpu.sync_copy(tab_hbm, tab_sh)

    cnt_w = BASE_CNT + jnp.where(wid < EXTRA, 1, 0)
    start_w = wid * BASE_CNT + jnp.minimum(wid, EXTRA)

    pltpu.sync_copy(deg_hbm.at[pl.ds(start_w * CH, BASE_CNT * CH)],
                    dega.at[pl.ds(0, BASE_CNT * CH)])

    @pl.when(wid < EXTRA)
    def _():
        pltpu.sync_copy(deg_hbm.at[pl.ds(start_w * CH + BASE_CNT * CH, CH)],
                        dega.at[pl.ds(BASE_CNT * CH, CH)])

    plsc.subcore_barrier()

    def issue(t, b, owait):
        @pl.when(t < cnt_w)
        def _():
            base = (start_w + t) * CH

            @pl.when(owait)
            def _():
                pltpu.make_async_copy(
                    gbs[b], out_hbm.at[pl.ds(0, CH), :], oss[b]).wait()

            pltpu.async_copy(tab_sh.at[dega.at[pl.ds(t * CH, G)]],
                             gbs[b].at[pl.ds(0, G), :], gss[b])
            pltpu.async_copy(tab_sh.at[dega.at[pl.ds(t * CH + G, G)]],
                             gbs[b].at[pl.ds(G, G), :], gss[b])
            pltpu.async_copy(x_hbm.at[pl.ds(base, CH), :], xbs[b], xss[b])

    def work(t, b):
        @pl.when(t < cnt_w)
        def _():
            base = (start_w + t) * CH
            pltpu.make_async_copy(tab_sh.at[dega.at[pl.ds(t * CH, G)]],
                                  gbs[b].at[pl.ds(0, G), :], gss[b]).wait()
            pltpu.make_async_copy(tab_sh.at[dega.at[pl.ds(t * CH + G, G)]],
                                  gbs[b].at[pl.ds(G, G), :], gss[b]).wait()
            pltpu.make_async_copy(
                x_hbm.at[pl.ds(base, CH), :], xbs[b], xss[b]).wait()

            def row_body(r, c2):
                for dcol in range(D // 16):
                    sl = pl.ds(dcol * 16, 16)
                    gbs[b][r, sl] = gbs[b][r, sl] + xbs[b][r, sl]
                return c2

            lax.fori_loop(0, CH, row_body, 0)
            pltpu.async_copy(gbs[b], out_hbm.at[pl.ds(base, CH), :], oss[b])

    issue(0, 0, False)

    def outer_body(i, carry):
        t0 = 2 * i
        issue(t0 + 1, 1, i >= 1)
        work(t0, 0)
        issue(t0 + 2, 0, True)
        work(t0 + 1, 1)
        return carry

    lax.fori_loop(0, ITERS // 2, outer_body, 0)

    pltpu.make_async_copy(gbs[0], out_hbm.at[pl.ds(0, CH), :], oss[0]).wait()
    pltpu.make_async_copy(gbs[1], out_hbm.at[pl.ds(0, CH), :], oss[1]).wait()


@jax.jit
def _run(x, degrees_i32, degree_table):
    kern = pl.kernel(
        _sc_body,
        out_type=jax.ShapeDtypeStruct((N, D), jnp.float32),
        mesh=plsc.VectorSubcoreMesh(core_axis_name="c", subcore_axis_name="s"),
        scratch_types=[
            pltpu.VMEM_SHARED((NUM_DEGREE, D), jnp.float32),
            pltpu.VMEM((ITERS * CH,), jnp.int32),
            pltpu.VMEM((CH, D), jnp.float32),
            pltpu.VMEM((CH, D), jnp.float32),
            pltpu.VMEM((CH, D), jnp.float32),
            pltpu.VMEM((CH, D), jnp.float32),
            pltpu.SemaphoreType.DMA,
            pltpu.SemaphoreType.DMA,
            pltpu.SemaphoreType.DMA,
            pltpu.SemaphoreType.DMA,
            pltpu.SemaphoreType.DMA,
            pltpu.SemaphoreType.DMA,
        ],
    )
    return kern(x, degrees_i32, degree_table)


def kernel(x, degrees, degree_table):
    return _run(x, degrees.astype(jnp.int32), degree_table)
